# baseline (speedup 1.0000x reference)
import functools

import jax
import jax.numpy as jnp
from jax import lax
from jax.experimental import pallas as pl
from jax.experimental.pallas import tpu as pltpu
from jax.experimental.pallas import tpu_sc as plsc

NUM_GRAPHS = 64
BM = 1024
CHUNK = 128
EB = 96


def _sc_mesh():
    return plsc.VectorSubcoreMesh(core_axis_name="c", subcore_axis_name="s")


def _sc_degree(dst_pad, n_pad):
    ep = dst_pad.shape[0]
    per_core = ep // 2
    per_tile = per_core // 16
    nbatch = per_tile // EB
    stripe = n_pad // 16

    @functools.partial(
        pl.kernel,
        out_type=jax.ShapeDtypeStruct((2, n_pad), jnp.float32),
        mesh=_sc_mesh(),
        scratch_types=[
            pltpu.VMEM((EB,), jnp.int32),
            pltpu.VMEM((EB,), jnp.float32),
            pltpu.VMEM((stripe,), jnp.float32),
            pltpu.VMEM_SHARED((n_pad,), jnp.float32),
        ],
    )
    def deg_kernel(dst_hbm, out_hbm, idx_v, ones_v, zero_v, acc):
        cid = lax.axis_index("c")
        sid = lax.axis_index("s")
        ones16 = jnp.full((16,), 1.0, jnp.float32)
        zero16 = jnp.zeros((16,), jnp.float32)
        for j in range(EB // 16):
            ones_v[pl.ds(j * 16, 16)] = ones16
        for j in range(stripe // 16):
            zero_v[pl.ds(j * 16, 16)] = zero16
        pltpu.sync_copy(zero_v, acc.at[pl.ds(sid * stripe, stripe)])
        plsc.subcore_barrier()

        def body(i, _):
            e0 = cid * per_core + sid * per_tile + i * EB
            pltpu.sync_copy(dst_hbm.at[pl.ds(e0, EB)], idx_v)
            pltpu.sync_copy(ones_v, acc.at[idx_v], add=True)
            return 0

        lax.fori_loop(0, nbatch, body, 0)
        plsc.subcore_barrier()
        pltpu.sync_copy(acc.at[pl.ds(sid * stripe, stripe)],
                        out_hbm.at[cid].at[pl.ds(sid * stripe, stripe)])

    return deg_kernel(dst_pad)


def _sc_scatter(g_flat, src_pad, dst_pad, zeros_hbm, n_pad, n_acc):
    ep = src_pad.shape[0]
    per_tile = ep // 16
    nbatch = per_tile // EB
    stripe = n_acc // 16

    @functools.partial(
        pl.kernel,
        out_type=jax.ShapeDtypeStruct((4, n_acc, CHUNK), jnp.float32),
        mesh=_sc_mesh(),
        scratch_types=[
            [pltpu.VMEM((EB,), jnp.int32) for _ in range(4)],
            [pltpu.VMEM((EB,), jnp.int32) for _ in range(4)],
            [pltpu.VMEM((EB, CHUNK), jnp.float32) for _ in range(4)],
            pltpu.VMEM_SHARED((n_acc, CHUNK), jnp.float32),
            [pltpu.SemaphoreType.DMA for _ in range(4)],
            [pltpu.SemaphoreType.DMA for _ in range(4)],
        ],
    )
    def scat_kernel(g_hbm, src_hbm, dst_hbm, z_hbm, out_hbm,
                    idx_g, idx_d, rows, acc, gsem, ssem):
        cid = lax.axis_index("c")
        sid = lax.axis_index("s")
        nvec = EB // 16

        def prep(j, b, off):
            e0 = sid * per_tile + j * EB
            pltpu.sync_copy(src_hbm.at[pl.ds(e0, EB)], idx_g[b])
            pltpu.sync_copy(dst_hbm.at[pl.ds(e0, EB)], idx_d[b])
            for v in range(nvec):
                sl = pl.ds(v * 16, 16)
                idx_g[b][sl] = idx_g[b][sl] + off

        def gather(b):
            pltpu.async_copy(g_hbm.at[idx_g[b]], rows[b], gsem[b])

        def wait_gather(b):
            pltpu.make_async_copy(g_hbm.at[idx_g[b]], rows[b], gsem[b]).wait()

        def scatter(b):
            pltpu.async_copy(rows[b], acc.at[idx_d[b]], ssem[b], add=True)

        def wait_scatter(b):
            pltpu.make_async_copy(rows[b], acc.at[idx_d[b]], ssem[b]).wait()

        def step(j, b, bn, off, first=False):
            wait_gather(b)
            if not first:
                wait_scatter(bn)
            scatter(b)
            prep(jnp.minimum(j + 2, nbatch - 1), bn, off)
            gather(bn)

        for k in range(2):
            chunk = cid * 2 + k
            off = chunk * n_pad
            pltpu.sync_copy(z_hbm, acc.at[pl.ds(sid * stripe, stripe)])
            plsc.subcore_barrier()

            prep(jnp.int32(0), 0, off)
            gather(0)
            prep(jnp.int32(1), 1, off)
            gather(1)
            step(jnp.int32(0), 0, 2, off, first=True)
            step(jnp.int32(1), 1, 3, off, first=True)

            def body(i, _):
                for c in range(4):
                    j = 4 * i + 2 + c
                    step(j, (2 + c) % 4, c % 4, off)
                return 0

            lax.fori_loop(0, (nbatch - 2) // 4, body, 0)
            wait_scatter((nbatch - 2) % 4)
            wait_scatter((nbatch - 1) % 4)
            wait_gather(nbatch % 4)
            wait_gather((nbatch + 1) % 4)
            plsc.subcore_barrier()
            pltpu.sync_copy(acc.at[pl.ds(sid * stripe, stripe)],
                            out_hbm.at[chunk].at[pl.ds(sid * stripe, stripe)])
            plsc.subcore_barrier()

    return scat_kernel(g_flat, src_pad, dst_pad, zeros_hbm)


def _dinv_block(degp_blk):
    ones = jnp.ones((2, 1), jnp.float32)
    deg = lax.dot_general(degp_blk, ones, (((0,), (0,)), ((), ())),
                          preferred_element_type=jnp.float32)
    return lax.rsqrt(deg + 1.0)


def _tc_first(x, w, degp, n_pad):
    d_in = x.shape[1]
    d_h = w.shape[1]
    grid = n_pad // BM

    def body(x_ref, w_ref, degp_ref, o_ref):
        dinv = _dinv_block(degp_ref[...])
        h = jnp.dot(x_ref[...], w_ref[...], preferred_element_type=jnp.float32)
        g = h * dinv
        for c in range(d_h // CHUNK):
            o_ref[c] = g[:, c * CHUNK:(c + 1) * CHUNK]

    return pl.pallas_call(
        body,
        grid=(grid,),
        in_specs=[
            pl.BlockSpec((BM, d_in), lambda i: (i, 0)),
            pl.BlockSpec((d_in, d_h), lambda i: (0, 0)),
            pl.BlockSpec((2, BM), lambda i: (0, i)),
        ],
        out_specs=pl.BlockSpec((d_h // CHUNK, BM, CHUNK), lambda i: (0, i, 0)),
        out_shape=jax.ShapeDtypeStruct((d_h // CHUNK, n_pad, CHUNK), jnp.float32),
    )(x, w, degp)


def _tc_mid(s, g, degp, b4, w4, n_pad):
    nc, _, _ = s.shape
    d_h = nc * CHUNK
    grid = n_pad // BM

    def body(s_ref, g_ref, degp_ref, b_ref, w_ref, o_ref):
        dinv = _dinv_block(degp_ref[...])
        acc = jnp.zeros((BM, d_h), jnp.float32)
        for c in range(nc):
            a = jnp.maximum((s_ref[c] + g_ref[c]) * dinv + b_ref[c], 0.0)
            acc += jnp.dot(a, w_ref[c], preferred_element_type=jnp.float32)
        gn = acc * dinv
        for c in range(nc):
            o_ref[c] = gn[:, c * CHUNK:(c + 1) * CHUNK]

    return pl.pallas_call(
        body,
        grid=(grid,),
        in_specs=[
            pl.BlockSpec((nc, BM, CHUNK), lambda i: (0, i, 0)),
            pl.BlockSpec((nc, BM, CHUNK), lambda i: (0, i, 0)),
            pl.BlockSpec((2, BM), lambda i: (0, i)),
            pl.BlockSpec((nc, 1, CHUNK), lambda i: (0, 0, 0)),
            pl.BlockSpec((nc, CHUNK, d_h), lambda i: (0, 0, 0)),
        ],
        out_specs=pl.BlockSpec((nc, BM, CHUNK), lambda i: (0, i, 0)),
        out_shape=jax.ShapeDtypeStruct((nc, n_pad, CHUNK), jnp.float32),
    )(s, g, degp, b4, w4)


def _tc_pool(s, g, degp, b4, batch2d, n, n_pad):
    nc, _, _ = s.shape
    d_h = nc * CHUNK
    grid = n_pad // BM

    def body(s_ref, g_ref, degp_ref, b_ref, batch_ref, o_ref):
        i = pl.program_id(0)
        dinv = _dinv_block(degp_ref[...])
        rows = i * BM + lax.broadcasted_iota(jnp.int32, (BM, 1), 0)
        valid = rows < n
        gids = lax.broadcasted_iota(jnp.int32, (1, NUM_GRAPHS), 1)
        pmat = jnp.where(batch_ref[...] == gids, 1.0, 0.0)

        @pl.when(i == 0)
        def _():
            o_ref[...] = jnp.zeros((NUM_GRAPHS, d_h), jnp.float32)

        for c in range(nc):
            pre = (s_ref[c] + g_ref[c]) * dinv + b_ref[c]
            sig = jnp.where(valid, jax.nn.sigmoid(pre), 0.0)
            part = lax.dot_general(pmat, sig, (((0,), (0,)), ((), ())),
                                   preferred_element_type=jnp.float32)
            o_ref[:, c * CHUNK:(c + 1) * CHUNK] += part

    return pl.pallas_call(
        body,
        grid=(grid,),
        in_specs=[
            pl.BlockSpec((nc, BM, CHUNK), lambda i: (0, i, 0)),
            pl.BlockSpec((nc, BM, CHUNK), lambda i: (0, i, 0)),
            pl.BlockSpec((2, BM), lambda i: (0, i)),
            pl.BlockSpec((nc, 1, CHUNK), lambda i: (0, 0, 0)),
            pl.BlockSpec((BM, 1), lambda i: (i, 0)),
        ],
        out_specs=pl.BlockSpec((NUM_GRAPHS, d_h), lambda i: (0, 0)),
        out_shape=jax.ShapeDtypeStruct((NUM_GRAPHS, d_h), jnp.float32),
    )(s, g, degp, b4, batch2d)


def kernel(x, edge_index, batch, W1, b1, W2, b2, W3, b3):
    n = x.shape[0]
    e = edge_index.shape[1]
    d_h = W1.shape[1]
    nc = d_h // CHUNK

    n_pad = ((n + BM - 1) // BM) * BM
    n_acc = (n + 1 + 127) // 128 * 128
    egran = 32 * EB
    e_pad = ((e + egran - 1) // egran) * egran
    while (e_pad // (16 * EB)) % 4 != 2:
        e_pad += egran

    pad = e_pad - e
    apad = jnp.arange(pad, dtype=jnp.int32)
    src_pad = jnp.concatenate([edge_index[0], apad % n])
    dst_pad = jnp.concatenate([edge_index[1], n + (apad % (n_acc - n))])

    degp = _sc_degree(dst_pad, n_pad)
    zeros_hbm = jnp.zeros((n_acc // 16, CHUNK), jnp.float32)

    b1r = b1.reshape(nc, 1, CHUNK)
    b2r = b2.reshape(nc, 1, CHUNK)
    b3r = b3.reshape(nc, 1, CHUNK)
    w2r = W2.reshape(nc, CHUNK, d_h)
    w3r = W3.reshape(nc, CHUNK, d_h)
    batch2d = batch.reshape(n, 1)

    g1 = _tc_first(x, W1, degp, n_pad)
    s1 = _sc_scatter(g1.reshape(nc * n_pad, CHUNK), src_pad, dst_pad,
                     zeros_hbm, n_pad, n_acc)
    g2 = _tc_mid(s1, g1, degp, b1r, w2r, n_pad)
    s2 = _sc_scatter(g2.reshape(nc * n_pad, CHUNK), src_pad, dst_pad,
                     zeros_hbm, n_pad, n_acc)
    g3 = _tc_mid(s2, g2, degp, b2r, w3r, n_pad)
    s3 = _sc_scatter(g3.reshape(nc * n_pad, CHUNK), src_pad, dst_pad,
                     zeros_hbm, n_pad, n_acc)
    return _tc_pool(s3, g3, degp, b3r, batch2d, n, n_pad)


# speedup vs baseline: 1.2222x; 1.2222x over previous
import functools

import jax
import jax.numpy as jnp
from jax import lax
from jax.experimental import pallas as pl
from jax.experimental.pallas import tpu as pltpu
from jax.experimental.pallas import tpu_sc as plsc

NUM_GRAPHS = 64
BM = 1024
CHUNK = 128
EB = 128


def _sc_mesh():
    return plsc.VectorSubcoreMesh(core_axis_name="c", subcore_axis_name="s")


def _sc_degree(dst_pad, n_pad):
    ep = dst_pad.shape[0]
    per_core = ep // 2
    per_tile = per_core // 16
    nbatch = per_tile // EB
    stripe = n_pad // 16

    @functools.partial(
        pl.kernel,
        out_type=jax.ShapeDtypeStruct((2, n_pad), jnp.float32),
        mesh=_sc_mesh(),
        scratch_types=[
            pltpu.VMEM((EB,), jnp.int32),
            pltpu.VMEM((EB,), jnp.float32),
            pltpu.VMEM((stripe,), jnp.float32),
            pltpu.VMEM_SHARED((n_pad,), jnp.float32),
        ],
    )
    def deg_kernel(dst_hbm, out_hbm, idx_v, ones_v, zero_v, acc):
        cid = lax.axis_index("c")
        sid = lax.axis_index("s")
        ones16 = jnp.full((16,), 1.0, jnp.float32)
        zero16 = jnp.zeros((16,), jnp.float32)
        for j in range(EB // 16):
            ones_v[pl.ds(j * 16, 16)] = ones16
        for j in range(stripe // 16):
            zero_v[pl.ds(j * 16, 16)] = zero16
        pltpu.sync_copy(zero_v, acc.at[pl.ds(sid * stripe, stripe)])
        plsc.subcore_barrier()

        def body(i, _):
            e0 = cid * per_core + sid * per_tile + i * EB
            pltpu.sync_copy(dst_hbm.at[pl.ds(e0, EB)], idx_v)
            pltpu.sync_copy(ones_v, acc.at[idx_v], add=True)
            return 0

        lax.fori_loop(0, nbatch, body, 0)
        plsc.subcore_barrier()
        pltpu.sync_copy(acc.at[pl.ds(sid * stripe, stripe)],
                        out_hbm.at[cid].at[pl.ds(sid * stripe, stripe)])

    return deg_kernel(dst_pad)


def _sc_scatter(g_flat, src_pad, dst_pad, zeros_hbm, n_pad, n_acc):
    ep = src_pad.shape[0]
    per_tile = ep // 16
    nbatch = per_tile // EB
    stripe = n_acc // 16

    @functools.partial(
        pl.kernel,
        out_type=jax.ShapeDtypeStruct((4, n_acc, CHUNK), jnp.float32),
        mesh=_sc_mesh(),
        scratch_types=[
            [pltpu.VMEM((EB,), jnp.int32) for _ in range(3)],
            [pltpu.VMEM((EB,), jnp.int32) for _ in range(3)],
            [pltpu.VMEM((EB, CHUNK), jnp.float32) for _ in range(3)],
            pltpu.VMEM_SHARED((n_acc, CHUNK), jnp.float32),
            [pltpu.SemaphoreType.DMA for _ in range(3)],
            [pltpu.SemaphoreType.DMA for _ in range(3)],
            pltpu.SemaphoreType.DMA,
        ],
    )
    def scat_kernel(g_hbm, src_hbm, dst_hbm, z_hbm, out_hbm,
                    idx_g, idx_d, rows, acc, gsem, ssem, psem):
        cid = lax.axis_index("c")
        sid = lax.axis_index("s")
        nvec = EB // 16

        def prep(j, b, off):
            e0 = sid * per_tile + j * EB
            d1 = pltpu.async_copy(src_hbm.at[pl.ds(e0, EB)], idx_g[b], psem)
            d2 = pltpu.async_copy(dst_hbm.at[pl.ds(e0, EB)], idx_d[b], psem)
            d1.wait()
            d2.wait()
            for v in range(nvec):
                sl = pl.ds(v * 16, 16)
                idx_g[b][sl] = idx_g[b][sl] + off

        def gather(b):
            pltpu.async_copy(g_hbm.at[idx_g[b]], rows[b], gsem[b])

        def wait_gather(b):
            pltpu.make_async_copy(g_hbm.at[idx_g[b]], rows[b], gsem[b]).wait()

        def scatter(b):
            pltpu.async_copy(rows[b], acc.at[idx_d[b]], ssem[b], add=True)

        def wait_scatter(b):
            pltpu.make_async_copy(rows[b], acc.at[idx_d[b]], ssem[b]).wait()

        def step(j, b, bn, off, first=False):
            wait_gather(b)
            if not first:
                wait_scatter(bn)
            scatter(b)
            prep(jnp.minimum(j + 2, nbatch - 1), bn, off)
            gather(bn)

        for k in range(2):
            chunk = cid * 2 + k
            off = chunk * n_pad
            pltpu.sync_copy(z_hbm, acc.at[pl.ds(sid * stripe, stripe)])
            plsc.subcore_barrier()

            prep(jnp.int32(0), 0, off)
            gather(0)
            prep(jnp.int32(1), 1, off)
            gather(1)
            step(jnp.int32(0), 0, 2, off, first=True)
            step(jnp.int32(1), 1, 0, off)

            def body(i, _):
                for c in range(3):
                    j = 3 * i + 2 + c
                    step(j, (2 + c) % 3, (1 + c) % 3, off)
                return 0

            lax.fori_loop(0, (nbatch - 2) // 3, body, 0)
            wait_scatter((nbatch - 1) % 3)
            wait_gather(nbatch % 3)
            wait_gather((nbatch + 1) % 3)
            plsc.subcore_barrier()
            pltpu.sync_copy(acc.at[pl.ds(sid * stripe, stripe)],
                            out_hbm.at[chunk].at[pl.ds(sid * stripe, stripe)])
            plsc.subcore_barrier()

    return scat_kernel(g_flat, src_pad, dst_pad, zeros_hbm)


def _dinv_block(degp_blk):
    ones = jnp.ones((2, 1), jnp.float32)
    deg = lax.dot_general(degp_blk, ones, (((0,), (0,)), ((), ())),
                          preferred_element_type=jnp.float32)
    return lax.rsqrt(deg + 1.0)


def _tc_first(x, w, degp, n_pad):
    d_in = x.shape[1]
    d_h = w.shape[1]
    grid = n_pad // BM

    def body(x_ref, w_ref, degp_ref, o_ref):
        dinv = _dinv_block(degp_ref[...])
        h = jnp.dot(x_ref[...], w_ref[...], preferred_element_type=jnp.float32)
        g = h * dinv
        for c in range(d_h // CHUNK):
            o_ref[c] = g[:, c * CHUNK:(c + 1) * CHUNK]

    return pl.pallas_call(
        body,
        grid=(grid,),
        in_specs=[
            pl.BlockSpec((BM, d_in), lambda i: (i, 0)),
            pl.BlockSpec((d_in, d_h), lambda i: (0, 0)),
            pl.BlockSpec((2, BM), lambda i: (0, i)),
        ],
        out_specs=pl.BlockSpec((d_h // CHUNK, BM, CHUNK), lambda i: (0, i, 0)),
        out_shape=jax.ShapeDtypeStruct((d_h // CHUNK, n_pad, CHUNK), jnp.float32),
    )(x, w, degp)


def _tc_mid(s, g, degp, b4, w4, n_pad):
    nc, _, _ = s.shape
    d_h = nc * CHUNK
    grid = n_pad // BM

    def body(s_ref, g_ref, degp_ref, b_ref, w_ref, o_ref):
        dinv = _dinv_block(degp_ref[...])
        acc = jnp.zeros((BM, d_h), jnp.float32)
        for c in range(nc):
            a = jnp.maximum((s_ref[c] + g_ref[c]) * dinv + b_ref[c], 0.0)
            acc += jnp.dot(a, w_ref[c], preferred_element_type=jnp.float32)
        gn = acc * dinv
        for c in range(nc):
            o_ref[c] = gn[:, c * CHUNK:(c + 1) * CHUNK]

    return pl.pallas_call(
        body,
        grid=(grid,),
        in_specs=[
            pl.BlockSpec((nc, BM, CHUNK), lambda i: (0, i, 0)),
            pl.BlockSpec((nc, BM, CHUNK), lambda i: (0, i, 0)),
            pl.BlockSpec((2, BM), lambda i: (0, i)),
            pl.BlockSpec((nc, 1, CHUNK), lambda i: (0, 0, 0)),
            pl.BlockSpec((nc, CHUNK, d_h), lambda i: (0, 0, 0)),
        ],
        out_specs=pl.BlockSpec((nc, BM, CHUNK), lambda i: (0, i, 0)),
        out_shape=jax.ShapeDtypeStruct((nc, n_pad, CHUNK), jnp.float32),
    )(s, g, degp, b4, w4)


def _tc_pool(s, g, degp, b4, batch2d, n, n_pad):
    nc, _, _ = s.shape
    d_h = nc * CHUNK
    grid = n_pad // BM

    def body(s_ref, g_ref, degp_ref, b_ref, batch_ref, o_ref):
        i = pl.program_id(0)
        dinv = _dinv_block(degp_ref[...])
        rows = i * BM + lax.broadcasted_iota(jnp.int32, (BM, 1), 0)
        valid = rows < n
        gids = lax.broadcasted_iota(jnp.int32, (1, NUM_GRAPHS), 1)
        pmat = jnp.where(batch_ref[...] == gids, 1.0, 0.0)

        @pl.when(i == 0)
        def _():
            o_ref[...] = jnp.zeros((NUM_GRAPHS, d_h), jnp.float32)

        for c in range(nc):
            pre = (s_ref[c] + g_ref[c]) * dinv + b_ref[c]
            sig = jnp.where(valid, jax.nn.sigmoid(pre), 0.0)
            part = lax.dot_general(pmat, sig, (((0,), (0,)), ((), ())),
                                   preferred_element_type=jnp.float32)
            o_ref[:, c * CHUNK:(c + 1) * CHUNK] += part

    return pl.pallas_call(
        body,
        grid=(grid,),
        in_specs=[
            pl.BlockSpec((nc, BM, CHUNK), lambda i: (0, i, 0)),
            pl.BlockSpec((nc, BM, CHUNK), lambda i: (0, i, 0)),
            pl.BlockSpec((2, BM), lambda i: (0, i)),
            pl.BlockSpec((nc, 1, CHUNK), lambda i: (0, 0, 0)),
            pl.BlockSpec((BM, 1), lambda i: (i, 0)),
        ],
        out_specs=pl.BlockSpec((NUM_GRAPHS, d_h), lambda i: (0, 0)),
        out_shape=jax.ShapeDtypeStruct((NUM_GRAPHS, d_h), jnp.float32),
    )(s, g, degp, b4, batch2d)


def kernel(x, edge_index, batch, W1, b1, W2, b2, W3, b3):
    n = x.shape[0]
    e = edge_index.shape[1]
    d_h = W1.shape[1]
    nc = d_h // CHUNK

    n_pad = ((n + BM - 1) // BM) * BM
    n_acc = (n + 1 + 127) // 128 * 128
    egran = 32 * EB
    e_pad = ((e + egran - 1) // egran) * egran
    while (e_pad // (16 * EB)) % 3 != 2:
        e_pad += egran

    pad = e_pad - e
    apad = jnp.arange(pad, dtype=jnp.int32)
    src_pad = jnp.concatenate([edge_index[0], apad % n])
    dst_pad = jnp.concatenate([edge_index[1], n + (apad % (n_acc - n))])

    degp = _sc_degree(dst_pad, n_pad)
    zeros_hbm = jnp.zeros((n_acc // 16, CHUNK), jnp.float32)

    b1r = b1.reshape(nc, 1, CHUNK)
    b2r = b2.reshape(nc, 1, CHUNK)
    b3r = b3.reshape(nc, 1, CHUNK)
    w2r = W2.reshape(nc, CHUNK, d_h)
    w3r = W3.reshape(nc, CHUNK, d_h)
    batch2d = batch.reshape(n, 1)

    g1 = _tc_first(x, W1, degp, n_pad)
    s1 = _sc_scatter(g1.reshape(nc * n_pad, CHUNK), src_pad, dst_pad,
                     zeros_hbm, n_pad, n_acc)
    g2 = _tc_mid(s1, g1, degp, b1r, w2r, n_pad)
    s2 = _sc_scatter(g2.reshape(nc * n_pad, CHUNK), src_pad, dst_pad,
                     zeros_hbm, n_pad, n_acc)
    g3 = _tc_mid(s2, g2, degp, b2r, w3r, n_pad)
    s3 = _sc_scatter(g3.reshape(nc * n_pad, CHUNK), src_pad, dst_pad,
                     zeros_hbm, n_pad, n_acc)
    return _tc_pool(s3, g3, degp, b3r, batch2d, n, n_pad)


# speedup vs baseline: 1.3577x; 1.1109x over previous
import functools

import jax
import jax.numpy as jnp
from jax import lax
from jax.experimental import pallas as pl
from jax.experimental.pallas import tpu as pltpu
from jax.experimental.pallas import tpu_sc as plsc

NUM_GRAPHS = 64
BM = 1024
CHUNK = 128
EB = 128


def _sc_mesh():
    return plsc.VectorSubcoreMesh(core_axis_name="c", subcore_axis_name="s")


def _sc_degree(dst_pad, n_pad):
    ep = dst_pad.shape[0]
    per_core = ep // 2
    per_tile = per_core // 16
    nbatch = per_tile // EB
    stripe = n_pad // 16

    @functools.partial(
        pl.kernel,
        out_type=jax.ShapeDtypeStruct((2, n_pad), jnp.float32),
        mesh=_sc_mesh(),
        scratch_types=[
            pltpu.VMEM((EB,), jnp.int32),
            pltpu.VMEM((EB,), jnp.float32),
            pltpu.VMEM((stripe,), jnp.float32),
            pltpu.VMEM_SHARED((n_pad,), jnp.float32),
        ],
    )
    def deg_kernel(dst_hbm, out_hbm, idx_v, ones_v, zero_v, acc):
        cid = lax.axis_index("c")
        sid = lax.axis_index("s")
        ones16 = jnp.full((16,), 1.0, jnp.float32)
        zero16 = jnp.zeros((16,), jnp.float32)
        for j in range(EB // 16):
            ones_v[pl.ds(j * 16, 16)] = ones16
        for j in range(stripe // 16):
            zero_v[pl.ds(j * 16, 16)] = zero16
        pltpu.sync_copy(zero_v, acc.at[pl.ds(sid * stripe, stripe)])
        plsc.subcore_barrier()

        def body(i, _):
            e0 = cid * per_core + sid * per_tile + i * EB
            pltpu.sync_copy(dst_hbm.at[pl.ds(e0, EB)], idx_v)
            pltpu.sync_copy(ones_v, acc.at[idx_v], add=True)
            return 0

        lax.fori_loop(0, nbatch, body, 0)
        plsc.subcore_barrier()
        pltpu.sync_copy(acc.at[pl.ds(sid * stripe, stripe)],
                        out_hbm.at[cid].at[pl.ds(sid * stripe, stripe)])

    return deg_kernel(dst_pad)


def _sc_scatter(g_flat, src_pad, dst_pad, zeros_hbm, n_pad, n_acc):
    ep = src_pad.shape[0]
    per_tile = ep // 16
    nbatch = per_tile // EB

    stripe = (n_acc // 16 + 7) // 8 * 8
    last_stripe = n_acc - 15 * stripe

    @functools.partial(
        pl.kernel,
        out_type=jax.ShapeDtypeStruct((4, n_acc, CHUNK), jnp.float32),
        mesh=_sc_mesh(),
        scratch_types=[
            [pltpu.VMEM((EB,), jnp.int32) for _ in range(6)],
            [pltpu.VMEM((EB,), jnp.int32) for _ in range(6)],
            [pltpu.VMEM((EB, CHUNK), jnp.float32) for _ in range(3)],
            pltpu.VMEM_SHARED((n_acc, CHUNK), jnp.float32),
            [pltpu.SemaphoreType.DMA for _ in range(3)],
            [pltpu.SemaphoreType.DMA for _ in range(3)],
            [pltpu.SemaphoreType.DMA for _ in range(6)],
        ],
    )
    def scat_kernel(g_hbm, src_hbm, dst_hbm, z_hbm, out_hbm,
                    idx_g, idx_d, rows, acc, gsem, ssem, isem):
        cid = lax.axis_index("c")
        sid = lax.axis_index("s")
        nvec = EB // 16

        def issue_idx(j, q):
            e0 = sid * per_tile + j * EB
            pltpu.async_copy(src_hbm.at[pl.ds(e0, EB)], idx_g[q], isem[q])
            pltpu.async_copy(dst_hbm.at[pl.ds(e0, EB)], idx_d[q], isem[q])

        def wait_idx(j, q, off, vadds=True):
            e0 = sid * per_tile + j * EB
            pltpu.make_async_copy(src_hbm.at[pl.ds(e0, EB)], idx_g[q],
                                  isem[q]).wait()
            pltpu.make_async_copy(dst_hbm.at[pl.ds(e0, EB)], idx_d[q],
                                  isem[q]).wait()
            if vadds:
                for v in range(nvec):
                    sl = pl.ds(v * 16, 16)
                    idx_g[q][sl] = idx_g[q][sl] + off

        def gather(b, q):
            pltpu.async_copy(g_hbm.at[idx_g[q]], rows[b], gsem[b])

        def wait_gather(b, q):
            pltpu.make_async_copy(g_hbm.at[idx_g[q]], rows[b], gsem[b]).wait()

        def scatter(b, q):
            pltpu.async_copy(rows[b], acc.at[idx_d[q]], ssem[b], add=True)

        def wait_scatter(b, q):
            pltpu.make_async_copy(rows[b], acc.at[idx_d[q]], ssem[b]).wait()

        def step(j, b, bn, q, qn, qf, off, first=False):
            wait_gather(b, q)
            if not first:
                wait_scatter(bn, (q + 5) % 6)
            scatter(b, q)
            wait_idx(jnp.minimum(j + 2, nbatch - 1), qn, off)
            gather(bn, qn)
            issue_idx(jnp.minimum(j + 4, nbatch - 1), qf)

        def zero_stripe():
            base = sid * stripe

            @pl.when(sid < 15)
            def _():
                pltpu.sync_copy(z_hbm, acc.at[pl.ds(base, stripe)])

            @pl.when(sid == 15)
            def _():
                pltpu.sync_copy(z_hbm.at[pl.ds(0, last_stripe)],
                                acc.at[pl.ds(base, last_stripe)])

        def copy_out(chunk):
            base = sid * stripe

            @pl.when(sid < 15)
            def _():
                pltpu.sync_copy(acc.at[pl.ds(base, stripe)],
                                out_hbm.at[chunk].at[pl.ds(base, stripe)])

            @pl.when(sid == 15)
            def _():
                pltpu.sync_copy(acc.at[pl.ds(base, last_stripe)],
                                out_hbm.at[chunk].at[pl.ds(base, last_stripe)])

        for k in range(2):
            chunk = cid * 2 + k
            off = chunk * n_pad
            zero_stripe()
            plsc.subcore_barrier()

            for q in range(4):
                issue_idx(jnp.int32(q), q)
            wait_idx(jnp.int32(0), 0, off)
            gather(0, 0)
            wait_idx(jnp.int32(1), 1, off)
            gather(1, 1)
            step(jnp.int32(0), 0, 2, 0, 2, 4, off, first=True)
            step(jnp.int32(1), 1, 0, 1, 3, 5, off)

            def body(i, _):
                for c in range(6):
                    j = 6 * i + 2 + c
                    step(j, (2 + c) % 3, (1 + c) % 3,
                         (2 + c) % 6, (4 + c) % 6, c % 6, off)
                return 0

            lax.fori_loop(0, (nbatch - 2) // 6, body, 0)
            wait_scatter((nbatch - 1) % 3, (nbatch - 1) % 6)
            wait_gather(nbatch % 3, nbatch % 6)
            wait_gather((nbatch + 1) % 3, (nbatch + 1) % 6)
            wait_idx(jnp.int32(nbatch - 1), (nbatch + 2) % 6, off, vadds=False)
            wait_idx(jnp.int32(nbatch - 1), (nbatch + 3) % 6, off, vadds=False)
            plsc.subcore_barrier()
            copy_out(chunk)
            plsc.subcore_barrier()

    return scat_kernel(g_flat, src_pad, dst_pad, zeros_hbm)


def _dinv_block(degp_blk):
    ones = jnp.ones((2, 1), jnp.float32)
    deg = lax.dot_general(degp_blk, ones, (((0,), (0,)), ((), ())),
                          preferred_element_type=jnp.float32)
    return lax.rsqrt(deg + 1.0)


def _tc_first(x, w, degp, n_pad):
    d_in = x.shape[1]
    d_h = w.shape[1]
    grid = n_pad // BM

    def body(x_ref, w_ref, degp_ref, o_ref):
        dinv = _dinv_block(degp_ref[...])
        h = jnp.dot(x_ref[...], w_ref[...], preferred_element_type=jnp.float32)
        g = h * dinv
        for c in range(d_h // CHUNK):
            o_ref[c] = g[:, c * CHUNK:(c + 1) * CHUNK]

    return pl.pallas_call(
        body,
        grid=(grid,),
        in_specs=[
            pl.BlockSpec((BM, d_in), lambda i: (i, 0)),
            pl.BlockSpec((d_in, d_h), lambda i: (0, 0)),
            pl.BlockSpec((2, BM), lambda i: (0, i)),
        ],
        out_specs=pl.BlockSpec((d_h // CHUNK, BM, CHUNK), lambda i: (0, i, 0)),
        out_shape=jax.ShapeDtypeStruct((d_h // CHUNK, n_pad, CHUNK), jnp.float32),
    )(x, w, degp)


def _tc_mid(s, g, degp, b4, w4, n_pad):
    nc, _, _ = s.shape
    d_h = nc * CHUNK
    grid = n_pad // BM

    def body(s_ref, g_ref, degp_ref, b_ref, w_ref, o_ref):
        dinv = _dinv_block(degp_ref[...])
        acc = jnp.zeros((BM, d_h), jnp.float32)
        for c in range(nc):
            a = jnp.maximum((s_ref[c] + g_ref[c]) * dinv + b_ref[c], 0.0)
            acc += jnp.dot(a, w_ref[c], preferred_element_type=jnp.float32)
        gn = acc * dinv
        for c in range(nc):
            o_ref[c] = gn[:, c * CHUNK:(c + 1) * CHUNK]

    return pl.pallas_call(
        body,
        grid=(grid,),
        in_specs=[
            pl.BlockSpec((nc, BM, CHUNK), lambda i: (0, i, 0)),
            pl.BlockSpec((nc, BM, CHUNK), lambda i: (0, i, 0)),
            pl.BlockSpec((2, BM), lambda i: (0, i)),
            pl.BlockSpec((nc, 1, CHUNK), lambda i: (0, 0, 0)),
            pl.BlockSpec((nc, CHUNK, d_h), lambda i: (0, 0, 0)),
        ],
        out_specs=pl.BlockSpec((nc, BM, CHUNK), lambda i: (0, i, 0)),
        out_shape=jax.ShapeDtypeStruct((nc, n_pad, CHUNK), jnp.float32),
    )(s, g, degp, b4, w4)


def _tc_pool(s, g, degp, b4, batch2d, n, n_pad):
    nc, _, _ = s.shape
    d_h = nc * CHUNK
    grid = n_pad // BM

    def body(s_ref, g_ref, degp_ref, b_ref, batch_ref, o_ref):
        i = pl.program_id(0)
        dinv = _dinv_block(degp_ref[...])
        rows = i * BM + lax.broadcasted_iota(jnp.int32, (BM, 1), 0)
        valid = rows < n
        gids = lax.broadcasted_iota(jnp.int32, (1, NUM_GRAPHS), 1)
        pmat = jnp.where(batch_ref[...] == gids, 1.0, 0.0)

        @pl.when(i == 0)
        def _():
            o_ref[...] = jnp.zeros((NUM_GRAPHS, d_h), jnp.float32)

        for c in range(nc):
            pre = (s_ref[c] + g_ref[c]) * dinv + b_ref[c]
            sig = jnp.where(valid, jax.nn.sigmoid(pre), 0.0)
            part = lax.dot_general(pmat, sig, (((0,), (0,)), ((), ())),
                                   preferred_element_type=jnp.float32)
            o_ref[:, c * CHUNK:(c + 1) * CHUNK] += part

    return pl.pallas_call(
        body,
        grid=(grid,),
        in_specs=[
            pl.BlockSpec((nc, BM, CHUNK), lambda i: (0, i, 0)),
            pl.BlockSpec((nc, BM, CHUNK), lambda i: (0, i, 0)),
            pl.BlockSpec((2, BM), lambda i: (0, i)),
            pl.BlockSpec((nc, 1, CHUNK), lambda i: (0, 0, 0)),
            pl.BlockSpec((BM, 1), lambda i: (i, 0)),
        ],
        out_specs=pl.BlockSpec((NUM_GRAPHS, d_h), lambda i: (0, 0)),
        out_shape=jax.ShapeDtypeStruct((NUM_GRAPHS, d_h), jnp.float32),
    )(s, g, degp, b4, batch2d)


def kernel(x, edge_index, batch, W1, b1, W2, b2, W3, b3):
    n = x.shape[0]
    e = edge_index.shape[1]
    d_h = W1.shape[1]
    nc = d_h // CHUNK

    n_pad = ((n + BM - 1) // BM) * BM
    n_acc = (n + 1 + 15) // 16 * 16
    egran = 32 * EB
    e_pad = ((e + egran - 1) // egran) * egran
    while (e_pad // (16 * EB)) % 6 != 2:
        e_pad += egran

    pad = e_pad - e
    apad = jnp.arange(pad, dtype=jnp.int32)
    src_pad = jnp.concatenate([edge_index[0], apad % n])
    dst_pad = jnp.concatenate([edge_index[1], n + (apad % (n_acc - n))])

    degp = _sc_degree(dst_pad, n_pad)
    zeros_hbm = jnp.zeros(((n_acc // 16 + 7) // 8 * 8, CHUNK), jnp.float32)

    b1r = b1.reshape(nc, 1, CHUNK)
    b2r = b2.reshape(nc, 1, CHUNK)
    b3r = b3.reshape(nc, 1, CHUNK)
    w2r = W2.reshape(nc, CHUNK, d_h)
    w3r = W3.reshape(nc, CHUNK, d_h)
    batch2d = batch.reshape(n, 1)

    g1 = _tc_first(x, W1, degp, n_pad)
    s1 = _sc_scatter(g1.reshape(nc * n_pad, CHUNK), src_pad, dst_pad,
                     zeros_hbm, n_pad, n_acc)
    g2 = _tc_mid(s1, g1, degp, b1r, w2r, n_pad)
    s2 = _sc_scatter(g2.reshape(nc * n_pad, CHUNK), src_pad, dst_pad,
                     zeros_hbm, n_pad, n_acc)
    g3 = _tc_mid(s2, g2, degp, b2r, w3r, n_pad)
    s3 = _sc_scatter(g3.reshape(nc * n_pad, CHUNK), src_pad, dst_pad,
                     zeros_hbm, n_pad, n_acc)
    return _tc_pool(s3, g3, degp, b3r, batch2d, n, n_pad)


# speedup vs baseline: 1.3964x; 1.0285x over previous
import functools

import jax
import jax.numpy as jnp
from jax import lax
from jax.experimental import pallas as pl
from jax.experimental.pallas import tpu as pltpu
from jax.experimental.pallas import tpu_sc as plsc

NUM_GRAPHS = 64
BM = 1024
CHUNK = 128
EB = 128


def _sc_mesh():
    return plsc.VectorSubcoreMesh(core_axis_name="c", subcore_axis_name="s")


def _sc_degree(dst2d, n_pad):
    nbt = dst2d.shape[0]
    nbd = nbt // 32
    stripe = n_pad // 16

    @functools.partial(
        pl.kernel,
        out_type=jax.ShapeDtypeStruct((2, n_pad), jnp.float32),
        mesh=_sc_mesh(),
        scratch_types=[
            pltpu.VMEM((nbd, EB), jnp.int32),
            pltpu.VMEM((EB,), jnp.float32),
            pltpu.VMEM((stripe,), jnp.float32),
            pltpu.VMEM_SHARED((n_pad,), jnp.float32),
            [pltpu.SemaphoreType.DMA for _ in range(2)],
        ],
    )
    def deg_kernel(dst_hbm, out_hbm, idx_all, ones_v, zero_v, acc, ssem):
        cid = lax.axis_index("c")
        sid = lax.axis_index("s")
        ones16 = jnp.full((16,), 1.0, jnp.float32)
        zero16 = jnp.zeros((16,), jnp.float32)
        for j in range(EB // 16):
            ones_v[pl.ds(j * 16, 16)] = ones16
        for j in range(stripe // 16):
            zero_v[pl.ds(j * 16, 16)] = zero16
        pltpu.sync_copy(zero_v, acc.at[pl.ds(sid * stripe, stripe)])
        base_b = (cid * 16 + sid) * nbd
        pltpu.sync_copy(dst_hbm.at[pl.ds(base_b, nbd)], idx_all)
        plsc.subcore_barrier()

        def scat(j, p):
            pltpu.async_copy(ones_v, acc.at[idx_all.at[j]], ssem[p], add=True)

        def wait_scat(j, p):
            pltpu.make_async_copy(ones_v, acc.at[idx_all.at[j]],
                                  ssem[p]).wait()

        scat(jnp.int32(0), 0)
        scat(jnp.int32(1), 1)

        def body(i, _):
            for c in range(2):
                j = 2 * i + 2 + c
                wait_scat(j - 2, c)
                scat(j, c)
            return 0

        lax.fori_loop(0, (nbd - 2) // 2, body, 0)
        wait_scat(jnp.int32(nbd - 2), 0)
        wait_scat(jnp.int32(nbd - 1), 1)
        plsc.subcore_barrier()
        pltpu.sync_copy(acc.at[pl.ds(sid * stripe, stripe)],
                        out_hbm.at[cid].at[pl.ds(sid * stripe, stripe)])

    return deg_kernel(dst2d)


def _sc_scatter(g_flat, src_pad, dst_pad, zeros_hbm, n_pad, n_acc):
    ep = src_pad.shape[0]
    per_tile = ep // 16
    nbatch = per_tile // EB

    stripe = (n_acc // 16 + 7) // 8 * 8
    last_stripe = n_acc - 15 * stripe

    @functools.partial(
        pl.kernel,
        out_type=jax.ShapeDtypeStruct((4, n_acc, CHUNK), jnp.float32),
        mesh=_sc_mesh(),
        scratch_types=[
            [pltpu.VMEM((EB,), jnp.int32) for _ in range(6)],
            [pltpu.VMEM((EB,), jnp.int32) for _ in range(6)],
            [pltpu.VMEM((EB, CHUNK), jnp.float32) for _ in range(3)],
            pltpu.VMEM_SHARED((n_acc, CHUNK), jnp.float32),
            [pltpu.SemaphoreType.DMA for _ in range(3)],
            [pltpu.SemaphoreType.DMA for _ in range(3)],
            [pltpu.SemaphoreType.DMA for _ in range(6)],
        ],
    )
    def scat_kernel(g_hbm, src_hbm, dst_hbm, z_hbm, out_hbm,
                    idx_g, idx_d, rows, acc, gsem, ssem, isem):
        cid = lax.axis_index("c")
        sid = lax.axis_index("s")
        nvec = EB // 16

        def issue_idx(j, q):
            e0 = sid * per_tile + j * EB
            pltpu.async_copy(src_hbm.at[pl.ds(e0, EB)], idx_g[q], isem[q])
            pltpu.async_copy(dst_hbm.at[pl.ds(e0, EB)], idx_d[q], isem[q])

        def wait_idx(j, q, off, vadds=True):
            e0 = sid * per_tile + j * EB
            pltpu.make_async_copy(src_hbm.at[pl.ds(e0, EB)], idx_g[q],
                                  isem[q]).wait()
            pltpu.make_async_copy(dst_hbm.at[pl.ds(e0, EB)], idx_d[q],
                                  isem[q]).wait()
            if vadds:
                for v in range(nvec):
                    sl = pl.ds(v * 16, 16)
                    idx_g[q][sl] = idx_g[q][sl] + off

        def gather(b, q):
            pltpu.async_copy(g_hbm.at[idx_g[q]], rows[b], gsem[b])

        def wait_gather(b, q):
            pltpu.make_async_copy(g_hbm.at[idx_g[q]], rows[b], gsem[b]).wait()

        def scatter(b, q):
            pltpu.async_copy(rows[b], acc.at[idx_d[q]], ssem[b], add=True)

        def wait_scatter(b, q):
            pltpu.make_async_copy(rows[b], acc.at[idx_d[q]], ssem[b]).wait()

        def step(j, b, bn, q, qn, qf, off, first=False):
            wait_gather(b, q)
            if not first:
                wait_scatter(bn, (q + 5) % 6)
            scatter(b, q)
            wait_idx(jnp.minimum(j + 2, nbatch - 1), qn, off)
            gather(bn, qn)
            issue_idx(jnp.minimum(j + 4, nbatch - 1), qf)

        def zero_stripe():
            base = sid * stripe

            @pl.when(sid < 15)
            def _():
                pltpu.sync_copy(z_hbm, acc.at[pl.ds(base, stripe)])

            @pl.when(sid == 15)
            def _():
                pltpu.sync_copy(z_hbm.at[pl.ds(0, last_stripe)],
                                acc.at[pl.ds(base, last_stripe)])

        def copy_out(chunk):
            base = sid * stripe

            @pl.when(sid < 15)
            def _():
                pltpu.sync_copy(acc.at[pl.ds(base, stripe)],
                                out_hbm.at[chunk].at[pl.ds(base, stripe)])

            @pl.when(sid == 15)
            def _():
                pltpu.sync_copy(acc.at[pl.ds(base, last_stripe)],
                                out_hbm.at[chunk].at[pl.ds(base, last_stripe)])

        for k in range(2):
            chunk = cid * 2 + k
            off = chunk * n_pad
            zero_stripe()
            plsc.subcore_barrier()

            for q in range(4):
                issue_idx(jnp.int32(q), q)
            wait_idx(jnp.int32(0), 0, off)
            gather(0, 0)
            wait_idx(jnp.int32(1), 1, off)
            gather(1, 1)
            step(jnp.int32(0), 0, 2, 0, 2, 4, off, first=True)
            step(jnp.int32(1), 1, 0, 1, 3, 5, off)

            def body(i, _):
                for c in range(6):
                    j = 6 * i + 2 + c
                    step(j, (2 + c) % 3, (1 + c) % 3,
                         (2 + c) % 6, (4 + c) % 6, c % 6, off)
                return 0

            lax.fori_loop(0, (nbatch - 2) // 6, body, 0)
            wait_scatter((nbatch - 1) % 3, (nbatch - 1) % 6)
            wait_gather(nbatch % 3, nbatch % 6)
            wait_gather((nbatch + 1) % 3, (nbatch + 1) % 6)
            wait_idx(jnp.int32(nbatch - 1), (nbatch + 2) % 6, off, vadds=False)
            wait_idx(jnp.int32(nbatch - 1), (nbatch + 3) % 6, off, vadds=False)
            plsc.subcore_barrier()
            copy_out(chunk)
            plsc.subcore_barrier()

    return scat_kernel(g_flat, src_pad, dst_pad, zeros_hbm)


def _dinv_block(degp_blk):
    ones = jnp.ones((2, 1), jnp.float32)
    deg = lax.dot_general(degp_blk, ones, (((0,), (0,)), ((), ())),
                          preferred_element_type=jnp.float32)
    return lax.rsqrt(deg + 1.0)


def _tc_first(x, w, degp, n_pad):
    d_in = x.shape[1]
    d_h = w.shape[1]
    grid = n_pad // BM

    def body(x_ref, w_ref, degp_ref, o_ref):
        dinv = _dinv_block(degp_ref[...])
        h = jnp.dot(x_ref[...], w_ref[...], preferred_element_type=jnp.float32)
        g = h * dinv
        for c in range(d_h // CHUNK):
            o_ref[c] = g[:, c * CHUNK:(c + 1) * CHUNK]

    return pl.pallas_call(
        body,
        grid=(grid,),
        in_specs=[
            pl.BlockSpec((BM, d_in), lambda i: (i, 0)),
            pl.BlockSpec((d_in, d_h), lambda i: (0, 0)),
            pl.BlockSpec((2, BM), lambda i: (0, i)),
        ],
        out_specs=pl.BlockSpec((d_h // CHUNK, BM, CHUNK), lambda i: (0, i, 0)),
        out_shape=jax.ShapeDtypeStruct((d_h // CHUNK, n_pad, CHUNK), jnp.float32),
    )(x, w, degp)


def _tc_mid(s, g, degp, b4, w4, n_pad):
    nc, _, _ = s.shape
    d_h = nc * CHUNK
    grid = n_pad // BM

    def body(s_ref, g_ref, degp_ref, b_ref, w_ref, o_ref):
        dinv = _dinv_block(degp_ref[...])
        acc = jnp.zeros((BM, d_h), jnp.float32)
        for c in range(nc):
            a = jnp.maximum((s_ref[c] + g_ref[c]) * dinv + b_ref[c], 0.0)
            acc += jnp.dot(a, w_ref[c], preferred_element_type=jnp.float32)
        gn = acc * dinv
        for c in range(nc):
            o_ref[c] = gn[:, c * CHUNK:(c + 1) * CHUNK]

    return pl.pallas_call(
        body,
        grid=(grid,),
        in_specs=[
            pl.BlockSpec((nc, BM, CHUNK), lambda i: (0, i, 0)),
            pl.BlockSpec((nc, BM, CHUNK), lambda i: (0, i, 0)),
            pl.BlockSpec((2, BM), lambda i: (0, i)),
            pl.BlockSpec((nc, 1, CHUNK), lambda i: (0, 0, 0)),
            pl.BlockSpec((nc, CHUNK, d_h), lambda i: (0, 0, 0)),
        ],
        out_specs=pl.BlockSpec((nc, BM, CHUNK), lambda i: (0, i, 0)),
        out_shape=jax.ShapeDtypeStruct((nc, n_pad, CHUNK), jnp.float32),
    )(s, g, degp, b4, w4)


def _tc_pool(s, g, degp, b4, batch2d, n, n_pad):
    nc, _, _ = s.shape
    d_h = nc * CHUNK
    grid = n_pad // BM

    def body(s_ref, g_ref, degp_ref, b_ref, batch_ref, o_ref):
        i = pl.program_id(0)
        dinv = _dinv_block(degp_ref[...])
        rows = i * BM + lax.broadcasted_iota(jnp.int32, (BM, 1), 0)
        valid = rows < n
        gids = lax.broadcasted_iota(jnp.int32, (1, NUM_GRAPHS), 1)
        pmat = jnp.where(batch_ref[...] == gids, 1.0, 0.0)

        @pl.when(i == 0)
        def _():
            o_ref[...] = jnp.zeros((NUM_GRAPHS, d_h), jnp.float32)

        for c in range(nc):
            pre = (s_ref[c] + g_ref[c]) * dinv + b_ref[c]
            sig = jnp.where(valid, jax.nn.sigmoid(pre), 0.0)
            part = lax.dot_general(pmat, sig, (((0,), (0,)), ((), ())),
                                   preferred_element_type=jnp.float32)
            o_ref[:, c * CHUNK:(c + 1) * CHUNK] += part

    return pl.pallas_call(
        body,
        grid=(grid,),
        in_specs=[
            pl.BlockSpec((nc, BM, CHUNK), lambda i: (0, i, 0)),
            pl.BlockSpec((nc, BM, CHUNK), lambda i: (0, i, 0)),
            pl.BlockSpec((2, BM), lambda i: (0, i)),
            pl.BlockSpec((nc, 1, CHUNK), lambda i: (0, 0, 0)),
            pl.BlockSpec((BM, 1), lambda i: (i, 0)),
        ],
        out_specs=pl.BlockSpec((NUM_GRAPHS, d_h), lambda i: (0, 0)),
        out_shape=jax.ShapeDtypeStruct((NUM_GRAPHS, d_h), jnp.float32),
    )(s, g, degp, b4, batch2d)


def kernel(x, edge_index, batch, W1, b1, W2, b2, W3, b3):
    n = x.shape[0]
    e = edge_index.shape[1]
    d_h = W1.shape[1]
    nc = d_h // CHUNK

    n_pad = ((n + BM - 1) // BM) * BM
    n_acc = (n + 1 + 15) // 16 * 16
    egran = 32 * EB
    e_pad = ((e + egran - 1) // egran) * egran
    while ((e_pad // (16 * EB)) % 6 != 2
           or (e_pad // (32 * EB)) % 2 != 0):
        e_pad += egran

    pad = e_pad - e
    apad = jnp.arange(pad, dtype=jnp.int32)
    src_pad = jnp.concatenate([edge_index[0], apad % n])
    dst_pad = jnp.concatenate([edge_index[1], n + (apad % (n_acc - n))])

    degp = _sc_degree(dst_pad.reshape(-1, EB), n_pad)
    zeros_hbm = jnp.zeros(((n_acc // 16 + 7) // 8 * 8, CHUNK), jnp.float32)

    b1r = b1.reshape(nc, 1, CHUNK)
    b2r = b2.reshape(nc, 1, CHUNK)
    b3r = b3.reshape(nc, 1, CHUNK)
    w2r = W2.reshape(nc, CHUNK, d_h)
    w3r = W3.reshape(nc, CHUNK, d_h)
    batch2d = batch.reshape(n, 1)

    g1 = _tc_first(x, W1, degp, n_pad)
    s1 = _sc_scatter(g1.reshape(nc * n_pad, CHUNK), src_pad, dst_pad,
                     zeros_hbm, n_pad, n_acc)
    g2 = _tc_mid(s1, g1, degp, b1r, w2r, n_pad)
    s2 = _sc_scatter(g2.reshape(nc * n_pad, CHUNK), src_pad, dst_pad,
                     zeros_hbm, n_pad, n_acc)
    g3 = _tc_mid(s2, g2, degp, b2r, w3r, n_pad)
    s3 = _sc_scatter(g3.reshape(nc * n_pad, CHUNK), src_pad, dst_pad,
                     zeros_hbm, n_pad, n_acc)
    return _tc_pool(s3, g3, degp, b3r, batch2d, n, n_pad)


# speedup vs baseline: 1.4072x; 1.0078x over previous
import functools

import jax
import jax.numpy as jnp
from jax import lax
from jax.experimental import pallas as pl
from jax.experimental.pallas import tpu as pltpu
from jax.experimental.pallas import tpu_sc as plsc

NUM_GRAPHS = 64
BM = 1024
CHUNK = 128
EB = 128


def _sc_mesh():
    return plsc.VectorSubcoreMesh(core_axis_name="c", subcore_axis_name="s")


def _sc_degree(dst2d, n_pad):
    nbt = dst2d.shape[0]
    nbd = nbt // 32
    stripe = n_pad // 16

    @functools.partial(
        pl.kernel,
        out_type=jax.ShapeDtypeStruct((2, n_pad), jnp.float32),
        mesh=_sc_mesh(),
        scratch_types=[
            pltpu.VMEM((nbd, EB), jnp.int32),
            pltpu.VMEM((EB,), jnp.float32),
            pltpu.VMEM((stripe,), jnp.float32),
            pltpu.VMEM_SHARED((n_pad,), jnp.float32),
            [pltpu.SemaphoreType.DMA for _ in range(2)],
        ],
    )
    def deg_kernel(dst_hbm, out_hbm, idx_all, ones_v, zero_v, acc, ssem):
        cid = lax.axis_index("c")
        sid = lax.axis_index("s")
        ones16 = jnp.full((16,), 1.0, jnp.float32)
        zero16 = jnp.zeros((16,), jnp.float32)
        for j in range(EB // 16):
            ones_v[pl.ds(j * 16, 16)] = ones16
        for j in range(stripe // 16):
            zero_v[pl.ds(j * 16, 16)] = zero16
        pltpu.sync_copy(zero_v, acc.at[pl.ds(sid * stripe, stripe)])
        base_b = (cid * 16 + sid) * nbd
        pltpu.sync_copy(dst_hbm.at[pl.ds(base_b, nbd)], idx_all)
        plsc.subcore_barrier()

        def scat(j, p):
            pltpu.async_copy(ones_v, acc.at[idx_all.at[j]], ssem[p], add=True)

        def wait_scat(j, p):
            pltpu.make_async_copy(ones_v, acc.at[idx_all.at[j]],
                                  ssem[p]).wait()

        scat(jnp.int32(0), 0)
        scat(jnp.int32(1), 1)

        def body(i, _):
            for c in range(2):
                j = 2 * i + 2 + c
                wait_scat(j - 2, c)
                scat(j, c)
            return 0

        lax.fori_loop(0, (nbd - 2) // 2, body, 0)
        wait_scat(jnp.int32(nbd - 2), 0)
        wait_scat(jnp.int32(nbd - 1), 1)
        plsc.subcore_barrier()
        pltpu.sync_copy(acc.at[pl.ds(sid * stripe, stripe)],
                        out_hbm.at[cid].at[pl.ds(sid * stripe, stripe)])

    return deg_kernel(dst2d)


def _sc_scatter(g_flat, src_pad, dst_pad, zeros_hbm, n_pad, n_acc):
    ep = src_pad.shape[0]
    per_tile = ep // 16
    nbatch = per_tile // EB

    stripe = (n_acc // 16 + 7) // 8 * 8
    last_stripe = n_acc - 15 * stripe

    @functools.partial(
        pl.kernel,
        out_type=jax.ShapeDtypeStruct((4, n_acc, CHUNK), jnp.float32),
        mesh=_sc_mesh(),
        scratch_types=[
            [pltpu.VMEM((EB,), jnp.int32) for _ in range(6)],
            [pltpu.VMEM((EB,), jnp.int32) for _ in range(6)],
            [pltpu.VMEM((EB, CHUNK), jnp.float32) for _ in range(3)],
            pltpu.VMEM_SHARED((n_acc, CHUNK), jnp.float32),
            [pltpu.SemaphoreType.DMA for _ in range(3)],
            [pltpu.SemaphoreType.DMA for _ in range(3)],
            [pltpu.SemaphoreType.DMA for _ in range(6)],
        ],
    )
    def scat_kernel(g_hbm, src_hbm, dst_hbm, z_hbm, out_hbm,
                    idx_g, idx_d, rows, acc, gsem, ssem, isem):
        cid = lax.axis_index("c")
        sid = lax.axis_index("s")
        nvec = EB // 16

        def issue_idx(j, q):
            e0 = sid * per_tile + j * EB
            pltpu.async_copy(src_hbm.at[pl.ds(e0, EB)], idx_g[q], isem[q])
            pltpu.async_copy(dst_hbm.at[pl.ds(e0, EB)], idx_d[q], isem[q])

        def wait_idx(j, q, off, vadds=True):
            e0 = sid * per_tile + j * EB
            pltpu.make_async_copy(src_hbm.at[pl.ds(e0, EB)], idx_g[q],
                                  isem[q]).wait()
            pltpu.make_async_copy(dst_hbm.at[pl.ds(e0, EB)], idx_d[q],
                                  isem[q]).wait()
            if vadds:
                for v in range(nvec):
                    sl = pl.ds(v * 16, 16)
                    idx_g[q][sl] = idx_g[q][sl] + off

        def gather(b, q):
            pltpu.async_copy(g_hbm.at[idx_g[q]], rows[b], gsem[b])

        def wait_gather(b, q):
            pltpu.make_async_copy(g_hbm.at[idx_g[q]], rows[b], gsem[b]).wait()

        def scatter(b, q):
            pltpu.async_copy(rows[b], acc.at[idx_d[q]], ssem[b], add=True)

        def wait_scatter(b, q):
            pltpu.make_async_copy(rows[b], acc.at[idx_d[q]], ssem[b]).wait()

        def step(j, b, bn, q, qn, qf, off, first=False):
            wait_gather(b, q)
            if not first:
                wait_scatter(bn, (q + 5) % 6)
            scatter(b, q)
            wait_idx(jnp.minimum(j + 2, nbatch - 1), qn, off)
            gather(bn, qn)
            issue_idx(jnp.minimum(j + 4, nbatch - 1), qf)

        def zero_stripe():
            base = sid * stripe

            @pl.when(sid < 15)
            def _():
                pltpu.sync_copy(z_hbm.at[pl.ds(base, stripe)],
                                acc.at[pl.ds(base, stripe)])

            @pl.when(sid == 15)
            def _():
                pltpu.sync_copy(z_hbm.at[pl.ds(base, last_stripe)],
                                acc.at[pl.ds(base, last_stripe)])

        def copy_out(chunk):
            base = sid * stripe

            @pl.when(sid < 15)
            def _():
                pltpu.sync_copy(acc.at[pl.ds(base, stripe)],
                                out_hbm.at[chunk].at[pl.ds(base, stripe)])

            @pl.when(sid == 15)
            def _():
                pltpu.sync_copy(acc.at[pl.ds(base, last_stripe)],
                                out_hbm.at[chunk].at[pl.ds(base, last_stripe)])

        for k in range(2):
            chunk = cid * 2 + k
            off = chunk * n_pad
            zero_stripe()
            plsc.subcore_barrier()

            for q in range(4):
                issue_idx(jnp.int32(q), q)
            wait_idx(jnp.int32(0), 0, off)
            gather(0, 0)
            wait_idx(jnp.int32(1), 1, off)
            gather(1, 1)
            step(jnp.int32(0), 0, 2, 0, 2, 4, off, first=True)
            step(jnp.int32(1), 1, 0, 1, 3, 5, off)

            def body(i, _):
                for c in range(6):
                    j = 6 * i + 2 + c
                    step(j, (2 + c) % 3, (1 + c) % 3,
                         (2 + c) % 6, (4 + c) % 6, c % 6, off)
                return 0

            lax.fori_loop(0, (nbatch - 2) // 6, body, 0)
            wait_scatter((nbatch - 1) % 3, (nbatch - 1) % 6)
            wait_gather(nbatch % 3, nbatch % 6)
            wait_gather((nbatch + 1) % 3, (nbatch + 1) % 6)
            wait_idx(jnp.int32(nbatch - 1), (nbatch + 2) % 6, off, vadds=False)
            wait_idx(jnp.int32(nbatch - 1), (nbatch + 3) % 6, off, vadds=False)
            plsc.subcore_barrier()
            copy_out(chunk)
            plsc.subcore_barrier()

    return scat_kernel(g_flat, src_pad, dst_pad, zeros_hbm)


def _dinv_block(degp_blk):
    ones = jnp.ones((2, 1), jnp.float32)
    deg = lax.dot_general(degp_blk, ones, (((0,), (0,)), ((), ())),
                          preferred_element_type=jnp.float32)
    return lax.rsqrt(deg + 1.0)


def _tc_first(x, w, degp, n_pad):
    d_in = x.shape[1]
    d_h = w.shape[1]
    grid = n_pad // BM

    def body(x_ref, w_ref, degp_ref, o_ref):
        dinv = _dinv_block(degp_ref[...])
        h = jnp.dot(x_ref[...], w_ref[...], preferred_element_type=jnp.float32)
        g = h * dinv
        for c in range(d_h // CHUNK):
            o_ref[c] = g[:, c * CHUNK:(c + 1) * CHUNK]

    return pl.pallas_call(
        body,
        grid=(grid,),
        in_specs=[
            pl.BlockSpec((BM, d_in), lambda i: (i, 0)),
            pl.BlockSpec((d_in, d_h), lambda i: (0, 0)),
            pl.BlockSpec((2, BM), lambda i: (0, i)),
        ],
        out_specs=pl.BlockSpec((d_h // CHUNK, BM, CHUNK), lambda i: (0, i, 0)),
        out_shape=jax.ShapeDtypeStruct((d_h // CHUNK, n_pad, CHUNK), jnp.float32),
    )(x, w, degp)


def _tc_mid(s, g, degp, b4, w4, n_pad):
    nc, _, _ = s.shape
    d_h = nc * CHUNK
    grid = n_pad // BM

    def body(s_ref, g_ref, degp_ref, b_ref, w_ref, o_ref):
        dinv = _dinv_block(degp_ref[...])
        acc = jnp.zeros((BM, d_h), jnp.float32)
        for c in range(nc):
            a = jnp.maximum((s_ref[c] + g_ref[c]) * dinv + b_ref[c], 0.0)
            acc += jnp.dot(a, w_ref[c], preferred_element_type=jnp.float32)
        gn = acc * dinv
        for c in range(nc):
            o_ref[c] = gn[:, c * CHUNK:(c + 1) * CHUNK]

    return pl.pallas_call(
        body,
        grid=(grid,),
        in_specs=[
            pl.BlockSpec((nc, BM, CHUNK), lambda i: (0, i, 0)),
            pl.BlockSpec((nc, BM, CHUNK), lambda i: (0, i, 0)),
            pl.BlockSpec((2, BM), lambda i: (0, i)),
            pl.BlockSpec((nc, 1, CHUNK), lambda i: (0, 0, 0)),
            pl.BlockSpec((nc, CHUNK, d_h), lambda i: (0, 0, 0)),
        ],
        out_specs=pl.BlockSpec((nc, BM, CHUNK), lambda i: (0, i, 0)),
        out_shape=jax.ShapeDtypeStruct((nc, n_pad, CHUNK), jnp.float32),
    )(s, g, degp, b4, w4)


def _tc_pool(s, g, degp, b4, batch2d, n, n_pad):
    nc, _, _ = s.shape
    d_h = nc * CHUNK
    grid = n_pad // BM

    def body(s_ref, g_ref, degp_ref, b_ref, batch_ref, o_ref):
        i = pl.program_id(0)
        dinv = _dinv_block(degp_ref[...])
        rows = i * BM + lax.broadcasted_iota(jnp.int32, (BM, 1), 0)
        valid = rows < n
        gids = lax.broadcasted_iota(jnp.int32, (1, NUM_GRAPHS), 1)
        pmat = jnp.where(batch_ref[...] == gids, 1.0, 0.0)

        @pl.when(i == 0)
        def _():
            o_ref[...] = jnp.zeros((NUM_GRAPHS, d_h), jnp.float32)

        for c in range(nc):
            pre = (s_ref[c] + g_ref[c]) * dinv + b_ref[c]
            sig = jnp.where(valid, jax.nn.sigmoid(pre), 0.0)
            part = lax.dot_general(pmat, sig, (((0,), (0,)), ((), ())),
                                   preferred_element_type=jnp.float32)
            o_ref[:, c * CHUNK:(c + 1) * CHUNK] += part

    return pl.pallas_call(
        body,
        grid=(grid,),
        in_specs=[
            pl.BlockSpec((nc, BM, CHUNK), lambda i: (0, i, 0)),
            pl.BlockSpec((nc, BM, CHUNK), lambda i: (0, i, 0)),
            pl.BlockSpec((2, BM), lambda i: (0, i)),
            pl.BlockSpec((nc, 1, CHUNK), lambda i: (0, 0, 0)),
            pl.BlockSpec((BM, 1), lambda i: (i, 0)),
        ],
        out_specs=pl.BlockSpec((NUM_GRAPHS, d_h), lambda i: (0, 0)),
        out_shape=jax.ShapeDtypeStruct((NUM_GRAPHS, d_h), jnp.float32),
    )(s, g, degp, b4, batch2d)


def kernel(x, edge_index, batch, W1, b1, W2, b2, W3, b3):
    n = x.shape[0]
    e = edge_index.shape[1]
    d_h = W1.shape[1]
    nc = d_h // CHUNK

    n_pad = ((n + BM - 1) // BM) * BM
    n_acc = (n + 1 + 15) // 16 * 16
    egran = 32 * EB
    e_pad = ((e + egran - 1) // egran) * egran
    while ((e_pad // (16 * EB)) % 6 != 2
           or (e_pad // (32 * EB)) % 2 != 0):
        e_pad += egran

    pad = e_pad - e
    apad = jnp.arange(pad, dtype=jnp.int32)
    src_pad = jnp.concatenate([edge_index[0], apad % n])
    dst_pad = jnp.concatenate([edge_index[1], n + (apad % (n_acc - n))])

    degp = _sc_degree(dst_pad.reshape(-1, EB), n_pad)
    zeros_hbm = jnp.zeros((16 * ((n_acc // 16 + 7) // 8 * 8), CHUNK),
                          jnp.float32)

    b1r = b1.reshape(nc, 1, CHUNK)
    b2r = b2.reshape(nc, 1, CHUNK)
    b3r = b3.reshape(nc, 1, CHUNK)
    w2r = W2.reshape(nc, CHUNK, d_h)
    w3r = W3.reshape(nc, CHUNK, d_h)
    batch2d = batch.reshape(n, 1)

    g1 = _tc_first(x, W1, degp, n_pad)
    s1 = _sc_scatter(g1.reshape(nc * n_pad, CHUNK), src_pad, dst_pad,
                     zeros_hbm, n_pad, n_acc)
    g2 = _tc_mid(s1, g1, degp, b1r, w2r, n_pad)
    s2 = _sc_scatter(g2.reshape(nc * n_pad, CHUNK), src_pad, dst_pad,
                     zeros_hbm, n_pad, n_acc)
    g3 = _tc_mid(s2, g2, degp, b2r, w3r, n_pad)
    s3 = _sc_scatter(g3.reshape(nc * n_pad, CHUNK), src_pad, dst_pad,
                     zeros_hbm, n_pad, n_acc)
    return _tc_pool(s3, g3, degp, b3r, batch2d, n, n_pad)


# speedup vs baseline: 1.4349x; 1.0197x over previous
import functools

import jax
import jax.numpy as jnp
from jax import lax
from jax.experimental import pallas as pl
from jax.experimental.pallas import tpu as pltpu
from jax.experimental.pallas import tpu_sc as plsc

NUM_GRAPHS = 64
BM = 1024
CHUNK = 128
EB = 128


def _sc_mesh():
    return plsc.VectorSubcoreMesh(core_axis_name="c", subcore_axis_name="s")


def _sc_degree(dst2d, n_pad):
    nbt = dst2d.shape[0]
    nbd = nbt // 32
    stripe = n_pad // 16

    @functools.partial(
        pl.kernel,
        out_type=jax.ShapeDtypeStruct((2, n_pad), jnp.float32),
        mesh=_sc_mesh(),
        scratch_types=[
            pltpu.VMEM((nbd, EB), jnp.int32),
            pltpu.VMEM((EB,), jnp.float32),
            pltpu.VMEM((stripe,), jnp.float32),
            pltpu.VMEM_SHARED((n_pad,), jnp.float32),
            [pltpu.SemaphoreType.DMA for _ in range(2)],
        ],
    )
    def deg_kernel(dst_hbm, out_hbm, idx_all, ones_v, zero_v, acc, ssem):
        cid = lax.axis_index("c")
        sid = lax.axis_index("s")
        ones16 = jnp.full((16,), 1.0, jnp.float32)
        zero16 = jnp.zeros((16,), jnp.float32)
        for j in range(EB // 16):
            ones_v[pl.ds(j * 16, 16)] = ones16
        for j in range(stripe // 16):
            zero_v[pl.ds(j * 16, 16)] = zero16
        pltpu.sync_copy(zero_v, acc.at[pl.ds(sid * stripe, stripe)])
        base_b = (cid * 16 + sid) * nbd
        pltpu.sync_copy(dst_hbm.at[pl.ds(base_b, nbd)], idx_all)
        plsc.subcore_barrier()

        def scat(j, p):
            pltpu.async_copy(ones_v, acc.at[idx_all.at[j]], ssem[p], add=True)

        def wait_scat(j, p):
            pltpu.make_async_copy(ones_v, acc.at[idx_all.at[j]],
                                  ssem[p]).wait()

        scat(jnp.int32(0), 0)
        scat(jnp.int32(1), 1)

        def body(i, _):
            for c in range(2):
                j = 2 * i + 2 + c
                wait_scat(j - 2, c)
                scat(j, c)
            return 0

        lax.fori_loop(0, (nbd - 2) // 2, body, 0)
        wait_scat(jnp.int32(nbd - 2), 0)
        wait_scat(jnp.int32(nbd - 1), 1)
        plsc.subcore_barrier()
        pltpu.sync_copy(acc.at[pl.ds(sid * stripe, stripe)],
                        out_hbm.at[cid].at[pl.ds(sid * stripe, stripe)])

    return deg_kernel(dst2d)


def _sc_scatter(g_flat, src_pad, dst_pad, zeros_hbm, n_pad, n_acc):
    ep = src_pad.shape[0]
    per_tile = ep // 16
    nbatch = per_tile // EB

    stripe = (n_acc // 16 + 7) // 8 * 8
    last_stripe = n_acc - 15 * stripe

    @functools.partial(
        pl.kernel,
        out_type=jax.ShapeDtypeStruct((4, n_acc, CHUNK), jnp.float32),
        mesh=_sc_mesh(),
        scratch_types=[
            [pltpu.VMEM((EB,), jnp.int32) for _ in range(6)],
            [pltpu.VMEM((EB,), jnp.int32) for _ in range(6)],
            [pltpu.VMEM((EB, CHUNK), jnp.float32) for _ in range(3)],
            pltpu.VMEM_SHARED((n_acc, CHUNK), jnp.float32),
            [pltpu.SemaphoreType.DMA for _ in range(3)],
            [pltpu.SemaphoreType.DMA for _ in range(3)],
            [pltpu.SemaphoreType.DMA for _ in range(6)],
        ],
    )
    def scat_kernel(g_hbm, src_hbm, dst_hbm, z_hbm, out_hbm,
                    idx_g, idx_d, rows, acc, gsem, ssem, isem):
        cid = lax.axis_index("c")
        sid = lax.axis_index("s")
        nvec = EB // 16

        def issue_idx(j, q):
            e0 = sid * per_tile + j * EB
            pltpu.async_copy(src_hbm.at[pl.ds(e0, EB)], idx_g[q], isem[q])
            pltpu.async_copy(dst_hbm.at[pl.ds(e0, EB)], idx_d[q], isem[q])

        def wait_idx(j, q, off, vadds=True):
            e0 = sid * per_tile + j * EB
            pltpu.make_async_copy(src_hbm.at[pl.ds(e0, EB)], idx_g[q],
                                  isem[q]).wait()
            pltpu.make_async_copy(dst_hbm.at[pl.ds(e0, EB)], idx_d[q],
                                  isem[q]).wait()
            if vadds:
                for v in range(nvec):
                    sl = pl.ds(v * 16, 16)
                    idx_g[q][sl] = idx_g[q][sl] + off

        def gather(b, q):
            pltpu.async_copy(g_hbm.at[idx_g[q]], rows[b], gsem[b])

        def wait_gather(b, q):
            pltpu.make_async_copy(g_hbm.at[idx_g[q]], rows[b], gsem[b]).wait()

        def scatter(b, q):
            pltpu.async_copy(rows[b], acc.at[idx_d[q]], ssem[b], add=True)

        def wait_scatter(b, q):
            pltpu.make_async_copy(rows[b], acc.at[idx_d[q]], ssem[b]).wait()

        def step(j, b, bn, q, qn, qf, off, first=False):
            wait_gather(b, q)
            if not first:
                wait_scatter(bn, (q + 5) % 6)
            scatter(b, q)
            wait_idx(jnp.minimum(j + 2, nbatch - 1), qn, off)
            gather(bn, qn)
            issue_idx(jnp.minimum(j + 4, nbatch - 1), qf)

        def zero_stripe():
            base = sid * stripe

            @pl.when(sid < 15)
            def _():
                pltpu.sync_copy(z_hbm.at[pl.ds(base, stripe)],
                                acc.at[pl.ds(base, stripe)])

            @pl.when(sid == 15)
            def _():
                pltpu.sync_copy(z_hbm.at[pl.ds(base, last_stripe)],
                                acc.at[pl.ds(base, last_stripe)])

        def copy_out(chunk):
            base = sid * stripe

            @pl.when(sid < 15)
            def _():
                pltpu.sync_copy(acc.at[pl.ds(base, stripe)],
                                out_hbm.at[chunk].at[pl.ds(base, stripe)])

            @pl.when(sid == 15)
            def _():
                pltpu.sync_copy(acc.at[pl.ds(base, last_stripe)],
                                out_hbm.at[chunk].at[pl.ds(base, last_stripe)])

        def prologue(off):
            for q in range(4):
                issue_idx(jnp.int32(q), q)
            wait_idx(jnp.int32(0), 0, off)
            gather(0, 0)
            wait_idx(jnp.int32(1), 1, off)
            gather(1, 1)

        def run_chunk(chunk, off):
            step(jnp.int32(0), 0, 2, 0, 2, 4, off, first=True)
            step(jnp.int32(1), 1, 0, 1, 3, 5, off)

            def body(i, _):
                for c in range(6):
                    j = 6 * i + 2 + c
                    step(j, (2 + c) % 3, (1 + c) % 3,
                         (2 + c) % 6, (4 + c) % 6, c % 6, off)
                return 0

            lax.fori_loop(0, (nbatch - 2) // 6, body, 0)
            wait_scatter((nbatch - 1) % 3, (nbatch - 1) % 6)
            wait_gather(nbatch % 3, nbatch % 6)
            wait_gather((nbatch + 1) % 3, (nbatch + 1) % 6)
            wait_idx(jnp.int32(nbatch - 1), (nbatch + 2) % 6, off, vadds=False)
            wait_idx(jnp.int32(nbatch - 1), (nbatch + 3) % 6, off, vadds=False)

        off0 = (cid * 2) * n_pad
        off1 = (cid * 2 + 1) * n_pad
        prologue(off0)
        zero_stripe()
        plsc.subcore_barrier()
        run_chunk(cid * 2, off0)
        prologue(off1)
        plsc.subcore_barrier()
        copy_out(cid * 2)
        plsc.subcore_barrier()
        zero_stripe()
        plsc.subcore_barrier()
        run_chunk(cid * 2 + 1, off1)
        plsc.subcore_barrier()
        copy_out(cid * 2 + 1)
        plsc.subcore_barrier()

    return scat_kernel(g_flat, src_pad, dst_pad, zeros_hbm)


def _dinv_block(degp_blk):
    ones = jnp.ones((2, 1), jnp.float32)
    deg = lax.dot_general(degp_blk, ones, (((0,), (0,)), ((), ())),
                          preferred_element_type=jnp.float32)
    return lax.rsqrt(deg + 1.0)


def _tc_first(x, w, degp, n_pad):
    d_in = x.shape[1]
    d_h = w.shape[1]
    grid = n_pad // BM

    def body(x_ref, w_ref, degp_ref, o_ref):
        dinv = _dinv_block(degp_ref[...])
        h = jnp.dot(x_ref[...], w_ref[...], preferred_element_type=jnp.float32)
        g = h * dinv
        for c in range(d_h // CHUNK):
            o_ref[c] = g[:, c * CHUNK:(c + 1) * CHUNK]

    return pl.pallas_call(
        body,
        grid=(grid,),
        in_specs=[
            pl.BlockSpec((BM, d_in), lambda i: (i, 0)),
            pl.BlockSpec((d_in, d_h), lambda i: (0, 0)),
            pl.BlockSpec((2, BM), lambda i: (0, i)),
        ],
        out_specs=pl.BlockSpec((d_h // CHUNK, BM, CHUNK), lambda i: (0, i, 0)),
        out_shape=jax.ShapeDtypeStruct((d_h // CHUNK, n_pad, CHUNK), jnp.float32),
    )(x, w, degp)


def _tc_mid(s, g, degp, b4, w4, n_pad):
    nc, _, _ = s.shape
    d_h = nc * CHUNK
    grid = n_pad // BM

    def body(s_ref, g_ref, degp_ref, b_ref, w_ref, o_ref):
        dinv = _dinv_block(degp_ref[...])
        acc = jnp.zeros((BM, d_h), jnp.float32)
        for c in range(nc):
            a = jnp.maximum((s_ref[c] + g_ref[c]) * dinv + b_ref[c], 0.0)
            acc += jnp.dot(a, w_ref[c], preferred_element_type=jnp.float32)
        gn = acc * dinv
        for c in range(nc):
            o_ref[c] = gn[:, c * CHUNK:(c + 1) * CHUNK]

    return pl.pallas_call(
        body,
        grid=(grid,),
        in_specs=[
            pl.BlockSpec((nc, BM, CHUNK), lambda i: (0, i, 0)),
            pl.BlockSpec((nc, BM, CHUNK), lambda i: (0, i, 0)),
            pl.BlockSpec((2, BM), lambda i: (0, i)),
            pl.BlockSpec((nc, 1, CHUNK), lambda i: (0, 0, 0)),
            pl.BlockSpec((nc, CHUNK, d_h), lambda i: (0, 0, 0)),
        ],
        out_specs=pl.BlockSpec((nc, BM, CHUNK), lambda i: (0, i, 0)),
        out_shape=jax.ShapeDtypeStruct((nc, n_pad, CHUNK), jnp.float32),
    )(s, g, degp, b4, w4)


def _tc_pool(s, g, degp, b4, batch2d, n, n_pad):
    nc, _, _ = s.shape
    d_h = nc * CHUNK
    grid = n_pad // BM

    def body(s_ref, g_ref, degp_ref, b_ref, batch_ref, o_ref):
        i = pl.program_id(0)
        dinv = _dinv_block(degp_ref[...])
        rows = i * BM + lax.broadcasted_iota(jnp.int32, (BM, 1), 0)
        valid = rows < n
        gids = lax.broadcasted_iota(jnp.int32, (1, NUM_GRAPHS), 1)
        pmat = jnp.where(batch_ref[...] == gids, 1.0, 0.0)

        @pl.when(i == 0)
        def _():
            o_ref[...] = jnp.zeros((NUM_GRAPHS, d_h), jnp.float32)

        for c in range(nc):
            pre = (s_ref[c] + g_ref[c]) * dinv + b_ref[c]
            sig = jnp.where(valid, jax.nn.sigmoid(pre), 0.0)
            part = lax.dot_general(pmat, sig, (((0,), (0,)), ((), ())),
                                   preferred_element_type=jnp.float32)
            o_ref[:, c * CHUNK:(c + 1) * CHUNK] += part

    return pl.pallas_call(
        body,
        grid=(grid,),
        in_specs=[
            pl.BlockSpec((nc, BM, CHUNK), lambda i: (0, i, 0)),
            pl.BlockSpec((nc, BM, CHUNK), lambda i: (0, i, 0)),
            pl.BlockSpec((2, BM), lambda i: (0, i)),
            pl.BlockSpec((nc, 1, CHUNK), lambda i: (0, 0, 0)),
            pl.BlockSpec((BM, 1), lambda i: (i, 0)),
        ],
        out_specs=pl.BlockSpec((NUM_GRAPHS, d_h), lambda i: (0, 0)),
        out_shape=jax.ShapeDtypeStruct((NUM_GRAPHS, d_h), jnp.float32),
    )(s, g, degp, b4, batch2d)


def kernel(x, edge_index, batch, W1, b1, W2, b2, W3, b3):
    n = x.shape[0]
    e = edge_index.shape[1]
    d_h = W1.shape[1]
    nc = d_h // CHUNK

    n_pad = ((n + BM - 1) // BM) * BM
    n_acc = (n + 1 + 15) // 16 * 16
    egran = 32 * EB
    e_pad = ((e + egran - 1) // egran) * egran
    while ((e_pad // (16 * EB)) % 6 != 2
           or (e_pad // (32 * EB)) % 2 != 0):
        e_pad += egran

    pad = e_pad - e
    apad = jnp.arange(pad, dtype=jnp.int32)
    src_pad = jnp.concatenate([edge_index[0], apad % n])
    dst_pad = jnp.concatenate([edge_index[1], n + (apad % (n_acc - n))])

    degp = _sc_degree(dst_pad.reshape(-1, EB), n_pad)
    zeros_hbm = jnp.zeros((16 * ((n_acc // 16 + 7) // 8 * 8), CHUNK),
                          jnp.float32)

    b1r = b1.reshape(nc, 1, CHUNK)
    b2r = b2.reshape(nc, 1, CHUNK)
    b3r = b3.reshape(nc, 1, CHUNK)
    w2r = W2.reshape(nc, CHUNK, d_h)
    w3r = W3.reshape(nc, CHUNK, d_h)
    batch2d = batch.reshape(n, 1)

    g1 = _tc_first(x, W1, degp, n_pad)
    s1 = _sc_scatter(g1.reshape(nc * n_pad, CHUNK), src_pad, dst_pad,
                     zeros_hbm, n_pad, n_acc)
    g2 = _tc_mid(s1, g1, degp, b1r, w2r, n_pad)
    s2 = _sc_scatter(g2.reshape(nc * n_pad, CHUNK), src_pad, dst_pad,
                     zeros_hbm, n_pad, n_acc)
    g3 = _tc_mid(s2, g2, degp, b2r, w3r, n_pad)
    s3 = _sc_scatter(g3.reshape(nc * n_pad, CHUNK), src_pad, dst_pad,
                     zeros_hbm, n_pad, n_acc)
    return _tc_pool(s3, g3, degp, b3r, batch2d, n, n_pad)
